# baseline (speedup 1.0000x reference)
import functools

import jax
import jax.numpy as jnp
from jax import lax
from jax.experimental import pallas as pl
from jax.experimental.pallas import tpu as pltpu
from jax.experimental.pallas import tpu_sc as plsc

N_NODES = 10000
N_EDGES = 320000
EMB = 128
HID = 100
HIDP = 128

NC = 2
NS = 16
NW = NC * NS

NODES_P = 10240
EDGES_P = 327680
EDGES_PER_TILE = EDGES_P // NW
CHUNK = 64
N_CHUNKS = EDGES_PER_TILE // CHUNK
BATCH = 16
NBATCH = N_CHUNKS // BATCH
ROWS_PER_TILE = NODES_P // NS
ROWS_PER_W = NODES_P // NW
GCH = 80

_mesh = plsc.VectorSubcoreMesh(core_axis_name="c", subcore_axis_name="s")


def _mm_body(emb_ref, w_ref, p_ref):
    p_ref[...] = jnp.dot(emb_ref[...], w_ref[...],
                         preferred_element_type=jnp.float32)


def _project(emb_table, w1p):
    m_blk = 1000
    return pl.pallas_call(
        _mm_body,
        grid=(N_NODES // m_blk,),
        in_specs=[
            pl.BlockSpec((m_blk, EMB), lambda i: (i, 0)),
            pl.BlockSpec((EMB, HIDP), lambda i: (0, 0)),
        ],
        out_specs=pl.BlockSpec((m_blk, HIDP), lambda i: (i, 0)),
        out_shape=jax.ShapeDtypeStruct((N_NODES, HIDP), jnp.float32),
    )(emb_table, w1p)


@functools.partial(
    pl.kernel,
    out_type=jax.ShapeDtypeStruct((NODES_P, HIDP), jnp.float32),
    mesh=_mesh,
    scratch_types=[
        pltpu.VMEM((GCH,), jnp.int32),
        pltpu.VMEM((GCH, HIDP), jnp.float32),
        pltpu.SemaphoreType.DMA,
    ],
)
def _sc_support(p_hbm, hn_hbm, sup_hbm, idx_v, rows_v, sem):
    wid = lax.axis_index("s") * NC + lax.axis_index("c")
    base = wid * ROWS_PER_W

    def chunk(i, carry):
        off = base + i * GCH
        pltpu.sync_copy(hn_hbm.at[pl.ds(off, GCH)], idx_v)
        pltpu.async_copy(p_hbm.at[idx_v], rows_v, sem).wait()
        pltpu.sync_copy(rows_v, sup_hbm.at[pl.ds(off, GCH)])
        return carry

    lax.fori_loop(0, ROWS_PER_W // GCH, chunk, 0)


NB = 4


def _bcast(gv, l):
    return lax.gather(
        gv, jnp.full((16, 1), l, jnp.int32),
        lax.GatherDimensionNumbers(offset_dims=(),
                                   collapsed_slice_dims=(0,),
                                   start_index_map=(0,)),
        (1,), mode=lax.GatherScatterMode.PROMISE_IN_BOUNDS)


@functools.partial(
    pl.kernel,
    out_type=jax.ShapeDtypeStruct((2 * NODES_P, HIDP), jnp.float32),
    mesh=_mesh,
    scratch_types=(
        [
            pltpu.VMEM((2, BATCH, CHUNK), jnp.int32),
            pltpu.VMEM((2, BATCH, CHUNK), jnp.int32),
            pltpu.VMEM((2, BATCH, CHUNK), jnp.float32),
        ]
        + [pltpu.VMEM((CHUNK, HIDP), jnp.float32)] * NB
        + [pltpu.VMEM_SHARED((NODES_P, HIDP), jnp.float32)]
        + [pltpu.SemaphoreType.DMA] * (2 * NB + 1)
    ),
)
def _sc_edges(src_hbm, dst_hbm, vals_hbm, sup_hbm, out_hbm,
              srcb, dstb, valsb, r0, r1, r2, r3, outsp,
              g0, g1, g2, g3, s0, s1, s2, s3, isem):
    rows = (r0, r1, r2, r3)
    gsem = (g0, g1, g2, g3)
    ssem = (s0, s1, s2, s3)
    cid = lax.axis_index("c")
    sid = lax.axis_index("s")
    wid = sid * NC + cid
    ibase = wid * NBATCH

    zero16 = jnp.zeros((16,), jnp.float32)

    def zrow(e, carry):
        for j in range(HIDP // 16):
            r0[e, pl.ds(j * 16, 16)] = zero16
        return carry

    lax.fori_loop(0, CHUNK, zrow, 0)

    def zcp(k, carry):
        pltpu.sync_copy(
            r0, outsp.at[pl.ds(sid * ROWS_PER_TILE + k * CHUNK, CHUNK)])
        return carry

    lax.fori_loop(0, ROWS_PER_TILE // CHUNK, zcp, 0)
    pltpu.sync_copy(outsp.at[pl.ds(sid * ROWS_PER_TILE, CHUNK)], r1)
    plsc.subcore_barrier()

    def issue_gather(p, lc, b):
        pltpu.async_copy(sup_hbm.at[srcb.at[p, lc]], rows[b], gsem[b])

    def wait_gather(b):
        pltpu.make_async_copy(sup_hbm.at[srcb.at[0, 0]], rows[b], gsem[b]).wait()

    def issue_scatter(p, lc, b):
        pltpu.async_copy(rows[b], outsp.at[dstb.at[p, lc]], ssem[b], add=True)

    def wait_scatter(b):
        pltpu.make_async_copy(rows[b], outsp.at[dstb.at[0, 0]], ssem[b]).wait()

    def issue_idx(k, pp):
        pltpu.async_copy(src_hbm.at[ibase + k], srcb.at[pp], isem)
        pltpu.async_copy(dst_hbm.at[ibase + k], dstb.at[pp], isem)
        pltpu.async_copy(vals_hbm.at[ibase + k], valsb.at[pp], isem)

    def wait_idx():
        for _ in range(3):
            pltpu.make_async_copy(src_hbm.at[0], srcb.at[0], isem).wait()

    def scale(p, lc, rref):
        @plsc.parallel_loop(0, CHUNK, step=1, unroll=4)
        def _body(e):
            g16 = (e // 16) * 16
            lane = e - g16
            gv = valsb[p, lc, pl.ds(g16, 16)]
            v = _bcast(gv, lane)
            for jj in range(HIDP // 16):
                sl = pl.ds(jj * 16, 16)
                rref[e, sl] = rref[e, sl] * v

    pltpu.sync_copy(src_hbm.at[ibase + NBATCH - 1], srcb.at[1])
    issue_gather(1, BATCH - 1, 0)

    pltpu.sync_copy(src_hbm.at[ibase], srcb.at[0])
    pltpu.sync_copy(dst_hbm.at[ibase], dstb.at[0])
    pltpu.sync_copy(vals_hbm.at[ibase], valsb.at[0])
    for lc in range(BATCH):
        ch = lc
        b = ch % NB
        bn = (b + 1) % NB
        if ch >= NB - 1:
            wait_scatter(bn)
        issue_gather(0, lc, bn)
        wait_gather(b)
        scale(0, lc, rows[b])
        issue_scatter(0, lc, b)
        if lc == NB - 1:
            issue_idx(1, 1)

    def batch_body(k, carry):
        p = k % 2
        wait_idx()

        def turn(q, j):
            lc = q * NB + j
            b = j
            bn = (b + 1) % NB
            wait_scatter(bn)
            issue_gather(p, lc, bn)
            wait_gather(b)
            scale(p, lc, rows[b])
            issue_scatter(p, lc, b)

        for j in range(NB):
            turn(0, j)
        issue_idx(k + 1, (k + 1) % 2)

        def qloop(q, c2):
            for j in range(NB):
                turn(q, j)
            return c2

        lax.fori_loop(1, BATCH // NB, qloop, 0)
        return carry

    lax.fori_loop(1, NBATCH - 1, batch_body, 0)

    pf = (NBATCH - 1) % 2
    wait_idx()
    for lc in range(BATCH):
        ch = (NBATCH - 1) * BATCH + lc
        b = ch % NB
        bn = (b + 1) % NB
        if ch < N_CHUNKS - 1:
            wait_scatter(bn)
            issue_gather(pf, lc, bn)
        wait_gather(b)
        scale(pf, lc, rows[b])
        issue_scatter(pf, lc, b)

    for b in range(NB):
        wait_scatter(b)

    plsc.subcore_barrier()

    tb = sid * ROWS_PER_TILE
    pltpu.sync_copy(outsp.at[pl.ds(tb, ROWS_PER_TILE)],
                    out_hbm.at[pl.ds(cid * NODES_P + tb, ROWS_PER_TILE)])


def _add_body(p0_ref, p1_ref, b_ref, o_ref):
    o_ref[...] = p0_ref[0] + p1_ref[0] + b_ref[...]


def _final_add(parts3, b1p):
    m_blk = 1000
    return pl.pallas_call(
        _add_body,
        grid=(N_NODES // m_blk,),
        in_specs=[
            pl.BlockSpec((1, m_blk, HIDP), lambda i: (0, i, 0)),
            pl.BlockSpec((1, m_blk, HIDP), lambda i: (1, i, 0)),
            pl.BlockSpec((1, HIDP), lambda i: (0, 0)),
        ],
        out_specs=pl.BlockSpec((m_blk, HIDP), lambda i: (i, 0)),
        out_shape=jax.ShapeDtypeStruct((N_NODES, HIDP), jnp.float32),
    )(parts3, parts3, b1p.reshape(1, HIDP))


def kernel(hyperneigh, adj_src, adj_dst, adj_vals, emb_table, W1, b1):
    hn = hyperneigh.astype(jnp.int32)
    src = adj_src.astype(jnp.int32)
    dst = adj_dst.astype(jnp.int32)
    vals = adj_vals.astype(jnp.float32)

    w1p = jnp.pad(W1, ((0, 0), (0, HIDP - HID)))
    b1p = jnp.pad(b1, (0, HIDP - HID))

    pad_n = NODES_P - N_NODES
    hn_p = jnp.concatenate([hn, (jnp.arange(pad_n, dtype=jnp.int32) * 37) % N_NODES])
    pad_e = EDGES_P - N_EDGES
    fill = (jnp.arange(pad_e, dtype=jnp.int32) * 131) % N_NODES
    shp = (NW * NBATCH, BATCH, CHUNK)
    src_p = jnp.roll(jnp.concatenate([src, fill]).reshape(NW, EDGES_PER_TILE),
                     -CHUNK, axis=1).reshape(shp)
    dst_p = jnp.concatenate([dst, fill]).reshape(shp)
    vals_p = jnp.concatenate(
        [vals, jnp.zeros((pad_e,), jnp.float32)]).reshape(shp)

    p = _project(emb_table, w1p)
    sup = _sc_support(p, hn_p)
    parts = _sc_edges(src_p, dst_p, vals_p, sup)
    out = _final_add(parts.reshape(2, NODES_P, HIDP), b1p)
    return out[:, :HID]


# speedup vs baseline: 1.0207x; 1.0207x over previous
import functools

import jax
import jax.numpy as jnp
from jax import lax
from jax.experimental import pallas as pl
from jax.experimental.pallas import tpu as pltpu
from jax.experimental.pallas import tpu_sc as plsc

N_NODES = 10000
N_EDGES = 320000
EMB = 128
HID = 100
HIDP = 128

NC = 2
NS = 16
NW = NC * NS

NODES_P = 10240
EDGES_P = 327680
EDGES_PER_TILE = EDGES_P // NW
CHUNK = 64
N_CHUNKS = EDGES_PER_TILE // CHUNK
BATCH = 16
NBATCH = N_CHUNKS // BATCH
ROWS_PER_TILE = NODES_P // NS
ROWS_PER_W = NODES_P // NW
GCH = 80

_mesh = plsc.VectorSubcoreMesh(core_axis_name="c", subcore_axis_name="s")


def _mm_body(emb_ref, w_ref, p_ref):
    p_ref[...] = jnp.dot(emb_ref[...], w_ref[...],
                         preferred_element_type=jnp.float32)


def _project(emb_table, w1p):
    m_blk = 1000
    return pl.pallas_call(
        _mm_body,
        grid=(N_NODES // m_blk,),
        in_specs=[
            pl.BlockSpec((m_blk, EMB), lambda i: (i, 0)),
            pl.BlockSpec((EMB, HIDP), lambda i: (0, 0)),
        ],
        out_specs=pl.BlockSpec((m_blk, HIDP), lambda i: (i, 0)),
        out_shape=jax.ShapeDtypeStruct((N_NODES, HIDP), jnp.float32),
    )(emb_table, w1p)


@functools.partial(
    pl.kernel,
    out_type=jax.ShapeDtypeStruct((NODES_P, HIDP), jnp.float32),
    mesh=_mesh,
    scratch_types=[
        pltpu.VMEM((ROWS_PER_W,), jnp.int32),
        pltpu.VMEM((ROWS_PER_W, HIDP), jnp.float32),
        pltpu.SemaphoreType.DMA,
    ],
)
def _sc_support(p_hbm, hn_hbm, sup_hbm, idx_v, rows_v, sem):
    wid = lax.axis_index("s") * NC + lax.axis_index("c")
    base = wid * ROWS_PER_W

    pltpu.sync_copy(hn_hbm.at[pl.ds(base, ROWS_PER_W)], idx_v)
    for k in range(ROWS_PER_W // GCH):
        pltpu.async_copy(p_hbm.at[idx_v.at[pl.ds(k * GCH, GCH)]],
                         rows_v.at[pl.ds(k * GCH, GCH)], sem)
    for k in range(ROWS_PER_W // GCH):
        pltpu.make_async_copy(
            p_hbm.at[idx_v.at[pl.ds(0, GCH)]],
            rows_v.at[pl.ds(k * GCH, GCH)], sem).wait()
    pltpu.sync_copy(rows_v, sup_hbm.at[pl.ds(base, ROWS_PER_W)])


NB = 4


def _bcast(gv, l):
    return lax.gather(
        gv, jnp.full((16, 1), l, jnp.int32),
        lax.GatherDimensionNumbers(offset_dims=(),
                                   collapsed_slice_dims=(0,),
                                   start_index_map=(0,)),
        (1,), mode=lax.GatherScatterMode.PROMISE_IN_BOUNDS)


@functools.partial(
    pl.kernel,
    out_type=jax.ShapeDtypeStruct((2 * NODES_P, HIDP), jnp.float32),
    mesh=_mesh,
    scratch_types=(
        [
            pltpu.VMEM((2, BATCH, CHUNK), jnp.int32),
            pltpu.VMEM((2, BATCH, CHUNK), jnp.int32),
            pltpu.VMEM((2, BATCH, CHUNK), jnp.float32),
        ]
        + [pltpu.VMEM((CHUNK, HIDP), jnp.float32)] * NB
        + [pltpu.VMEM_SHARED((NODES_P, HIDP), jnp.float32)]
        + [pltpu.SemaphoreType.DMA] * (2 * NB + 1)
    ),
)
def _sc_edges(src_hbm, dst_hbm, vals_hbm, sup_hbm, out_hbm,
              srcb, dstb, valsb, r0, r1, r2, r3, outsp,
              g0, g1, g2, g3, s0, s1, s2, s3, isem):
    rows = (r0, r1, r2, r3)
    gsem = (g0, g1, g2, g3)
    ssem = (s0, s1, s2, s3)
    cid = lax.axis_index("c")
    sid = lax.axis_index("s")
    wid = sid * NC + cid
    ibase = wid * NBATCH

    zero16 = jnp.zeros((16,), jnp.float32)

    def zrow(e, carry):
        for j in range(HIDP // 16):
            r0[e, pl.ds(j * 16, 16)] = zero16
        return carry

    lax.fori_loop(0, CHUNK, zrow, 0)

    def zcp(k, carry):
        pltpu.sync_copy(
            r0, outsp.at[pl.ds(sid * ROWS_PER_TILE + k * CHUNK, CHUNK)])
        return carry

    lax.fori_loop(0, ROWS_PER_TILE // CHUNK, zcp, 0)
    pltpu.sync_copy(outsp.at[pl.ds(sid * ROWS_PER_TILE, CHUNK)], r1)
    plsc.subcore_barrier()

    def issue_gather(p, lc, b):
        pltpu.async_copy(sup_hbm.at[srcb.at[p, lc]], rows[b], gsem[b])

    def wait_gather(b):
        pltpu.make_async_copy(sup_hbm.at[srcb.at[0, 0]], rows[b], gsem[b]).wait()

    def issue_scatter(p, lc, b):
        pltpu.async_copy(rows[b], outsp.at[dstb.at[p, lc]], ssem[b], add=True)

    def wait_scatter(b):
        pltpu.make_async_copy(rows[b], outsp.at[dstb.at[0, 0]], ssem[b]).wait()

    def issue_idx(k, pp):
        pltpu.async_copy(src_hbm.at[ibase + k], srcb.at[pp], isem)
        pltpu.async_copy(dst_hbm.at[ibase + k], dstb.at[pp], isem)
        pltpu.async_copy(vals_hbm.at[ibase + k], valsb.at[pp], isem)

    def wait_idx():
        for _ in range(3):
            pltpu.make_async_copy(src_hbm.at[0], srcb.at[0], isem).wait()

    def scale(p, lc, rref):
        @plsc.parallel_loop(0, CHUNK, step=1, unroll=4)
        def _body(e):
            g16 = (e // 16) * 16
            lane = e - g16
            gv = valsb[p, lc, pl.ds(g16, 16)]
            v = _bcast(gv, lane)
            for jj in range(HIDP // 16):
                sl = pl.ds(jj * 16, 16)
                rref[e, sl] = rref[e, sl] * v

    pltpu.sync_copy(src_hbm.at[ibase + NBATCH - 1], srcb.at[1])
    issue_gather(1, BATCH - 1, 0)

    pltpu.sync_copy(src_hbm.at[ibase], srcb.at[0])
    pltpu.sync_copy(dst_hbm.at[ibase], dstb.at[0])
    pltpu.sync_copy(vals_hbm.at[ibase], valsb.at[0])
    for lc in range(BATCH):
        ch = lc
        b = ch % NB
        bn = (b + 1) % NB
        if ch >= NB - 1:
            wait_scatter(bn)
        issue_gather(0, lc, bn)
        wait_gather(b)
        scale(0, lc, rows[b])
        issue_scatter(0, lc, b)
        if lc == NB - 1:
            issue_idx(1, 1)

    def batch_body(k, carry):
        p = k % 2
        wait_idx()

        def turn(q, j):
            lc = q * NB + j
            b = j
            bn = (b + 1) % NB
            wait_scatter(bn)
            issue_gather(p, lc, bn)
            wait_gather(b)
            scale(p, lc, rows[b])
            issue_scatter(p, lc, b)

        for j in range(NB):
            turn(0, j)
        issue_idx(k + 1, (k + 1) % 2)

        def qloop(q, c2):
            for j in range(NB):
                turn(q, j)
            return c2

        lax.fori_loop(1, BATCH // NB, qloop, 0)
        return carry

    lax.fori_loop(1, NBATCH - 1, batch_body, 0)

    pf = (NBATCH - 1) % 2
    wait_idx()
    for lc in range(BATCH):
        ch = (NBATCH - 1) * BATCH + lc
        b = ch % NB
        bn = (b + 1) % NB
        if ch < N_CHUNKS - 1:
            wait_scatter(bn)
            issue_gather(pf, lc, bn)
        wait_gather(b)
        scale(pf, lc, rows[b])
        issue_scatter(pf, lc, b)

    for b in range(NB):
        wait_scatter(b)

    plsc.subcore_barrier()

    tb = sid * ROWS_PER_TILE
    pltpu.sync_copy(outsp.at[pl.ds(tb, ROWS_PER_TILE)],
                    out_hbm.at[pl.ds(cid * NODES_P + tb, ROWS_PER_TILE)])


def _add_body(p0_ref, p1_ref, b_ref, o_ref):
    o_ref[...] = p0_ref[0] + p1_ref[0] + b_ref[...]


def _final_add(parts3, b1p):
    m_blk = 1000
    return pl.pallas_call(
        _add_body,
        grid=(N_NODES // m_blk,),
        in_specs=[
            pl.BlockSpec((1, m_blk, HIDP), lambda i: (0, i, 0)),
            pl.BlockSpec((1, m_blk, HIDP), lambda i: (1, i, 0)),
            pl.BlockSpec((1, HIDP), lambda i: (0, 0)),
        ],
        out_specs=pl.BlockSpec((m_blk, HIDP), lambda i: (i, 0)),
        out_shape=jax.ShapeDtypeStruct((N_NODES, HIDP), jnp.float32),
    )(parts3, parts3, b1p.reshape(1, HIDP))


def kernel(hyperneigh, adj_src, adj_dst, adj_vals, emb_table, W1, b1):
    hn = hyperneigh.astype(jnp.int32)
    src = adj_src.astype(jnp.int32)
    dst = adj_dst.astype(jnp.int32)
    vals = adj_vals.astype(jnp.float32)

    w1p = jnp.pad(W1, ((0, 0), (0, HIDP - HID)))
    b1p = jnp.pad(b1, (0, HIDP - HID))

    pad_n = NODES_P - N_NODES
    hn_p = jnp.concatenate([hn, (jnp.arange(pad_n, dtype=jnp.int32) * 37) % N_NODES])
    pad_e = EDGES_P - N_EDGES
    fill = (jnp.arange(pad_e, dtype=jnp.int32) * 131) % N_NODES
    shp = (NW * NBATCH, BATCH, CHUNK)
    src_p = jnp.roll(jnp.concatenate([src, fill]).reshape(NW, EDGES_PER_TILE),
                     -CHUNK, axis=1).reshape(shp)
    dst_p = jnp.concatenate([dst, fill]).reshape(shp)
    vals_p = jnp.concatenate(
        [vals, jnp.zeros((pad_e,), jnp.float32)]).reshape(shp)

    p = _project(emb_table, w1p)
    sup = _sc_support(p, hn_p)
    parts = _sc_edges(src_p, dst_p, vals_p, sup)
    out = _final_add(parts.reshape(2, NODES_P, HIDP), b1p)
    return out[:, :HID]
